# baseline (speedup 1.0000x reference)
import functools

import jax
import jax.numpy as jnp
from jax import lax
from jax.experimental import pallas as pl
from jax.experimental.pallas import tpu as pltpu
from jax.experimental.pallas import tpu_sc as plsc

_EPS = 1e-5
_L = 16


def _build_sc_call(n_tok, hidden, out_dtype):
  NW = 32
  TPW = n_tok // NW
  C = 16
  NCHUNK = TPW // C
  J32 = hidden // 32
  half = hidden // 2

  mesh = plsc.VectorSubcoreMesh(core_axis_name="c", subcore_axis_name="s")

  def body(ids_hbm, table_hbm, we_hbm, wo_hbm, out_hbm,
           idx_v, we_v, wo_v, rows0, rows1, ob0, ob1, scale_v,
           gsem0, gsem1, osem0, osem1):
    cid = lax.axis_index("c")
    sid = lax.axis_index("s")
    wid = sid * 2 + cid
    base = wid * TPW

    pltpu.sync_copy(ids_hbm.at[pl.ds(base, TPW)], idx_v)
    pltpu.sync_copy(we_hbm, we_v)
    pltpu.sync_copy(wo_hbm, wo_v)

    rows = (rows0, rows1)
    obs = (ob0, ob1)
    gsems = (gsem0, gsem1)
    osems = (osem0, osem1)

    def start_gather(g, b):
      pltpu.async_copy(table_hbm.at[idx_v.at[pl.ds(g * C, C)]], rows[b],
                       gsems[b])

    def wait_gather(b):
      pltpu.make_async_copy(table_hbm.at[idx_v.at[pl.ds(0, C)]], rows[b],
                            gsems[b]).wait()

    ii2 = 2 * lax.iota(jnp.int32, _L)

    def compute(rv, ob):
      def row_body(r, _):
        def ss_body(j, accs):
          a0, a1, a2, a3 = accs
          cbase = j * 64
          v0 = rv[r, pl.ds(cbase, _L)]
          v1 = rv[r, pl.ds(cbase + 16, _L)]
          v2 = rv[r, pl.ds(cbase + 32, _L)]
          v3 = rv[r, pl.ds(cbase + 48, _L)]
          return (a0 + v0 * v0, a1 + v1 * v1, a2 + v2 * v2, a3 + v3 * v3)

        z = jnp.zeros((_L,), jnp.float32)
        a0, a1, a2, a3 = lax.fori_loop(0, hidden // 64, ss_body,
                                       (z, z, z, z))
        s = jnp.sum((a0 + a1) + (a2 + a3))
        m = s * (1.0 / hidden) + _EPS
        i = lax.bitcast_convert_type(m, jnp.int32)
        i = 0x5F3759DF - lax.shift_right_arithmetic(i, 1)
        y = lax.bitcast_convert_type(i, jnp.float32)
        y = y * (1.5 - 0.5 * m * y * y)
        y = y * (1.5 - 0.5 * m * y * y)
        y = y * (1.5 - 0.5 * m * y * y)
        scale_v[r] = y
        return 0

      lax.fori_loop(0, C, row_body, 0)

      def col_body(j, _):
        ce = j * 32 + ii2
        co = ce + 1
        wej = we_v[pl.ds(j * _L, _L)]
        woj = wo_v[pl.ds(j * _L, _L)]
        for r in range(C):
          ys = scale_v[r]
          ridx = jnp.full((_L,), r, jnp.int32)
          a = plsc.load_gather(rv, [ridx, ce])
          b = plsc.load_gather(rv, [ridx, co])
          pa = (a * ys) * wej
          pb = (b * ys) * woj
          packed = plsc.pack(pa, pb, format=plsc.PackFormat.INTERLEAVED)
          ob[r, pl.ds(j * _L, _L)] = plsc.bitcast(packed, jnp.int32)
        return 0

      lax.fori_loop(0, J32, col_body, 0)

    start_gather(0, 0)

    def chunk_body(k, carry):
      for b in (0, 1):
        g = 2 * k + b
        wait_gather(b)

        @pl.when(g + 1 < NCHUNK)
        def _():
          start_gather(g + 1, 1 - b)

        @pl.when(g >= 2)
        def _():
          pltpu.make_async_copy(obs[b], out_hbm.at[pl.ds(base, C)],
                                osems[b]).wait()

        compute(rows[b], obs[b])
        pltpu.async_copy(obs[b], out_hbm.at[pl.ds(base + g * C, C)],
                         osems[b])
      return carry

    lax.fori_loop(0, NCHUNK // 2, chunk_body, 0)
    pltpu.make_async_copy(ob0, out_hbm.at[pl.ds(base, C)], osem0).wait()
    pltpu.make_async_copy(ob1, out_hbm.at[pl.ds(base, C)], osem1).wait()

  return pl.kernel(
      body,
      out_type=jax.ShapeDtypeStruct((n_tok, hidden // 2), jnp.int32),
      mesh=mesh,
      compiler_params=pltpu.CompilerParams(needs_layout_passes=False),
      scratch_types=[
          pltpu.VMEM((TPW,), jnp.int32),
          pltpu.VMEM((half,), jnp.float32),
          pltpu.VMEM((half,), jnp.float32),
          pltpu.VMEM((C, hidden), jnp.float32),
          pltpu.VMEM((C, hidden), jnp.float32),
          pltpu.VMEM((C, half), jnp.int32),
          pltpu.VMEM((C, half), jnp.int32),
          pltpu.SMEM((C,), jnp.float32),
          pltpu.SemaphoreType.DMA,
          pltpu.SemaphoreType.DMA,
          pltpu.SemaphoreType.DMA,
          pltpu.SemaphoreType.DMA,
      ],
  )


@functools.partial(jax.jit, static_argnames=())
def kernel(input_ids, tok_emb, norm_weight):
  b, s = input_ids.shape
  vocab, hidden = tok_emb.shape
  ids = input_ids.reshape(-1).astype(jnp.int32)
  we = norm_weight[0::2]
  wo = norm_weight[1::2]
  call = _build_sc_call(b * s, hidden, jnp.bfloat16)
  out = call(ids, tok_emb, we, wo)
  return lax.bitcast_convert_type(out, jnp.bfloat16).reshape(b, s, hidden)


# speedup vs baseline: 2.1694x; 2.1694x over previous
import functools

import jax
import jax.numpy as jnp
from jax import lax
from jax.experimental import pallas as pl
from jax.experimental.pallas import tpu as pltpu
from jax.experimental.pallas import tpu_sc as plsc

_EPS = 1e-5
_L = 16


def _build_sc_call(n_tok, hidden, out_dtype):
  NW = 32
  TPW = n_tok // NW
  C = 16
  NCHUNK = TPW // C
  J32 = hidden // 32
  half = hidden // 2

  mesh = plsc.VectorSubcoreMesh(core_axis_name="c", subcore_axis_name="s")

  def body(ids_hbm, table_hbm, we_hbm, wo_hbm, out_hbm,
           idx_v, we_v, wo_v, rows0, rows1, ob0, ob1, scale_v,
           gsem0, gsem1, osem0, osem1):
    cid = lax.axis_index("c")
    sid = lax.axis_index("s")
    wid = sid * 2 + cid
    base = wid * TPW

    pltpu.sync_copy(ids_hbm.at[pl.ds(base, TPW)], idx_v)
    pltpu.sync_copy(we_hbm, we_v)
    pltpu.sync_copy(wo_hbm, wo_v)

    rows = (rows0, rows1)
    obs = (ob0, ob1)
    gsems = (gsem0, gsem1)
    osems = (osem0, osem1)

    def start_gather(g, b):
      pltpu.async_copy(table_hbm.at[idx_v.at[pl.ds(g * C, C)]], rows[b],
                       gsems[b])

    def wait_gather(b):
      pltpu.make_async_copy(table_hbm.at[idx_v.at[pl.ds(0, C)]], rows[b],
                            gsems[b]).wait()

    ii2 = 2 * lax.iota(jnp.int32, _L)

    def compute(rv, ob):
      def row_body(r, _):
        def ss_body(j, accs):
          a0, a1, a2, a3 = accs
          cbase = j * 64
          v0 = rv[r, pl.ds(cbase, _L)]
          v1 = rv[r, pl.ds(cbase + 16, _L)]
          v2 = rv[r, pl.ds(cbase + 32, _L)]
          v3 = rv[r, pl.ds(cbase + 48, _L)]
          return (a0 + v0 * v0, a1 + v1 * v1, a2 + v2 * v2, a3 + v3 * v3)

        z = jnp.zeros((_L,), jnp.float32)
        a0, a1, a2, a3 = lax.fori_loop(0, hidden // 64, ss_body,
                                       (z, z, z, z))
        s = jnp.sum((a0 + a1) + (a2 + a3))
        m = s * (1.0 / hidden) + _EPS
        i = lax.bitcast_convert_type(m, jnp.int32)
        i = 0x5F3759DF - lax.shift_right_arithmetic(i, 1)
        y = lax.bitcast_convert_type(i, jnp.float32)
        y = y * (1.5 - 0.5 * m * y * y)
        y = y * (1.5 - 0.5 * m * y * y)
        y = y * (1.5 - 0.5 * m * y * y)
        scale_v[r] = y
        return 0

      lax.fori_loop(0, C, row_body, 0)

      for j in range(J32):
        ce = j * 32 + ii2
        co = ce + 1
        wej = we_v[pl.ds(j * _L, _L)]
        woj = wo_v[pl.ds(j * _L, _L)]

        def row2_body(r, carry, ce=ce, co=co, wej=wej, woj=woj, j=j):
          ys = scale_v[r]
          ridx = jnp.broadcast_to(r, (_L,))
          a = plsc.load_gather(rv, [ridx, ce])
          b = plsc.load_gather(rv, [ridx, co])
          pa = (a * ys) * wej
          pb = (b * ys) * woj
          packed = plsc.pack(pa, pb, format=plsc.PackFormat.INTERLEAVED)
          ob[r, pl.ds(j * 32, 32)] = packed
          return carry

        lax.fori_loop(0, C, row2_body, 0)

    start_gather(0, 0)

    def chunk_body(k, carry):
      for b in (0, 1):
        g = 2 * k + b
        wait_gather(b)

        @pl.when(g + 1 < NCHUNK)
        def _():
          start_gather(g + 1, 1 - b)

        @pl.when(g >= 2)
        def _():
          pltpu.make_async_copy(obs[b], out_hbm.at[pl.ds(base, C)],
                                osems[b]).wait()

        compute(rows[b], obs[b])
        pltpu.async_copy(obs[b], out_hbm.at[pl.ds(base + g * C, C)],
                         osems[b])
      return carry

    lax.fori_loop(0, NCHUNK // 2, chunk_body, 0)
    pltpu.make_async_copy(ob0, out_hbm.at[pl.ds(base, C)], osem0).wait()
    pltpu.make_async_copy(ob1, out_hbm.at[pl.ds(base, C)], osem1).wait()

  return pl.kernel(
      body,
      out_type=jax.ShapeDtypeStruct((n_tok, hidden), out_dtype),
      mesh=mesh,
      compiler_params=pltpu.CompilerParams(needs_layout_passes=False),
      scratch_types=[
          pltpu.VMEM((TPW,), jnp.int32),
          pltpu.VMEM((half,), jnp.float32),
          pltpu.VMEM((half,), jnp.float32),
          pltpu.VMEM((C, hidden), jnp.float32),
          pltpu.VMEM((C, hidden), jnp.float32),
          pltpu.VMEM((C, hidden), jnp.bfloat16),
          pltpu.VMEM((C, hidden), jnp.bfloat16),
          pltpu.SMEM((C,), jnp.float32),
          pltpu.SemaphoreType.DMA,
          pltpu.SemaphoreType.DMA,
          pltpu.SemaphoreType.DMA,
          pltpu.SemaphoreType.DMA,
      ],
  )


@functools.partial(jax.jit, static_argnames=())
def kernel(input_ids, tok_emb, norm_weight):
  b, s = input_ids.shape
  vocab, hidden = tok_emb.shape
  ids = input_ids.reshape(-1).astype(jnp.int32)
  we = norm_weight[0::2]
  wo = norm_weight[1::2]
  call = _build_sc_call(b * s, hidden, jnp.bfloat16)
  out = call(ids, tok_emb, we, wo)
  return out.reshape(b, s, hidden)


# speedup vs baseline: 4.5129x; 2.0802x over previous
import functools

import jax
import jax.numpy as jnp
from jax import lax
from jax.experimental import pallas as pl
from jax.experimental.pallas import tpu as pltpu
from jax.experimental.pallas import tpu_sc as plsc

_EPS = 1e-5
_L = 16


def _build_sc_call(n_tok, hidden, out_dtype):
  NW = 32
  TPW = n_tok // NW
  C = 16
  NCHUNK = TPW // C
  J16 = hidden // _L

  mesh = plsc.VectorSubcoreMesh(core_axis_name="c", subcore_axis_name="s")

  def body(ids_hbm, table_hbm, w_hbm, out_hbm,
           idx_v, w_v, rows0, rows1, ob0, ob1, scale_v,
           gsem0, gsem1, osem0, osem1):
    cid = lax.axis_index("c")
    sid = lax.axis_index("s")
    wid = sid * 2 + cid
    base = wid * TPW

    pltpu.sync_copy(ids_hbm.at[pl.ds(base, TPW)], idx_v)
    pltpu.sync_copy(w_hbm, w_v)

    rows = (rows0, rows1)
    obs = (ob0, ob1)
    gsems = (gsem0, gsem1)
    osems = (osem0, osem1)

    def start_gather(g, b):
      pltpu.async_copy(table_hbm.at[idx_v.at[pl.ds(g * C, C)]], rows[b],
                       gsems[b])

    def wait_gather(b):
      pltpu.make_async_copy(table_hbm.at[idx_v.at[pl.ds(0, C)]], rows[b],
                            gsems[b]).wait()

    def compute(rv, ob):
      def row_body(r, _):
        def ss_body(j, accs):
          a0, a1, a2, a3 = accs
          cbase = j * 64
          v0 = rv[r, pl.ds(cbase, _L)]
          v1 = rv[r, pl.ds(cbase + 16, _L)]
          v2 = rv[r, pl.ds(cbase + 32, _L)]
          v3 = rv[r, pl.ds(cbase + 48, _L)]
          return (a0 + v0 * v0, a1 + v1 * v1, a2 + v2 * v2, a3 + v3 * v3)

        z = jnp.zeros((_L,), jnp.float32)
        a0, a1, a2, a3 = lax.fori_loop(0, hidden // 64, ss_body,
                                       (z, z, z, z))
        s = jnp.sum((a0 + a1) + (a2 + a3))
        m = s * (1.0 / hidden) + _EPS
        i = lax.bitcast_convert_type(m, jnp.int32)
        i = 0x5F3759DF - lax.shift_right_arithmetic(i, 1)
        y = lax.bitcast_convert_type(i, jnp.float32)
        y = y * (1.5 - 0.5 * m * y * y)
        y = y * (1.5 - 0.5 * m * y * y)
        y = y * (1.5 - 0.5 * m * y * y)
        scale_v[r] = y
        return 0

      lax.fori_loop(0, C, row_body, 0)

      ys = [scale_v[r] for r in range(C)]

      def col_body(j, _):
        wj = w_v[pl.ds(j * _L, _L)]
        for p in range(C // 2):
          a = rv[2 * p, pl.ds(j * _L, _L)] * ys[2 * p]
          b = rv[2 * p + 1, pl.ds(j * _L, _L)] * ys[2 * p + 1]
          packed = plsc.pack(a * wj, b * wj,
                             format=plsc.PackFormat.INTERLEAVED)
          ob[p, pl.ds(j * _L, _L)] = plsc.bitcast(packed, jnp.int32)
        return 0

      lax.fori_loop(0, J16, col_body, 0)

    start_gather(0, 0)

    def chunk_body(k, carry):
      for b in (0, 1):
        g = 2 * k + b
        wait_gather(b)

        @pl.when(g + 1 < NCHUNK)
        def _():
          start_gather(g + 1, 1 - b)

        @pl.when(g >= 2)
        def _():
          pltpu.make_async_copy(obs[b].bitcast(out_dtype),
                                out_hbm.at[pl.ds(base, C)],
                                osems[b]).wait()

        compute(rows[b], obs[b])
        pltpu.async_copy(obs[b].bitcast(out_dtype),
                         out_hbm.at[pl.ds(base + g * C, C)],
                         osems[b])
      return carry

    lax.fori_loop(0, NCHUNK // 2, chunk_body, 0)
    pltpu.make_async_copy(ob0.bitcast(out_dtype),
                          out_hbm.at[pl.ds(base, C)], osem0).wait()
    pltpu.make_async_copy(ob1.bitcast(out_dtype),
                          out_hbm.at[pl.ds(base, C)], osem1).wait()

  return pl.kernel(
      body,
      out_type=jax.ShapeDtypeStruct((n_tok, hidden), out_dtype),
      mesh=mesh,
      compiler_params=pltpu.CompilerParams(needs_layout_passes=False),
      scratch_types=[
          pltpu.VMEM((TPW,), jnp.int32),
          pltpu.VMEM((hidden,), jnp.float32),
          pltpu.VMEM((C, hidden), jnp.float32),
          pltpu.VMEM((C, hidden), jnp.float32),
          pltpu.VMEM((C // 2, hidden), jnp.int32),
          pltpu.VMEM((C // 2, hidden), jnp.int32),
          pltpu.SMEM((C,), jnp.float32),
          pltpu.SemaphoreType.DMA,
          pltpu.SemaphoreType.DMA,
          pltpu.SemaphoreType.DMA,
          pltpu.SemaphoreType.DMA,
      ],
  )


@functools.partial(jax.jit, static_argnames=())
def kernel(input_ids, tok_emb, norm_weight):
  b, s = input_ids.shape
  vocab, hidden = tok_emb.shape
  ids = input_ids.reshape(-1).astype(jnp.int32)
  call = _build_sc_call(b * s, hidden, jnp.bfloat16)
  out = call(ids, tok_emb, norm_weight)
  return out.reshape(b, s, hidden)
